# baseline (speedup 1.0000x reference)
import functools

import jax
import jax.numpy as jnp
from jax import lax
from jax.experimental import pallas as pl
from jax.experimental.pallas import tpu as pltpu
from jax.experimental.pallas import tpu_sc as plsc

N = 10000
NFEAT = 128
NHID = 64
NCLASS = 64
NHEADS = 4
ALPHA = 0.2

BN = 2000
H1W = 136
H2W = 72


def _elu(z):
    return jnp.where(z > 0, z, jnp.exp(z) - 1.0)


def _mm1_body(x_ref, w_ref, a_ref, h_ref, f_ref):
    c = pl.program_id(0)
    xb = x_ref[...]
    cols = [jnp.dot(w_ref[j], a_ref[j, :NHID],
                    preferred_element_type=jnp.float32) for j in range(NHEADS)]
    cols += [jnp.dot(w_ref[j], a_ref[j, NHID:],
                     preferred_element_type=jnp.float32) for j in range(NHEADS)]
    wfull = jnp.concatenate(
        [w_ref[2 * c], w_ref[2 * c + 1], jnp.stack(cols, axis=1)], axis=1)
    hb = jnp.dot(xb, wfull, preferred_element_type=jnp.float32)
    h_ref[...] = hb
    f_ref[...] = hb[:, 128:]


def _mm1(x, W, a):
    return pl.pallas_call(
        _mm1_body,
        grid=(2, N // BN),
        in_specs=[
            pl.BlockSpec((BN, NFEAT), lambda c, i: (i, 0)),
            pl.BlockSpec((NHEADS, NFEAT, NHID), lambda c, i: (0, 0, 0)),
            pl.BlockSpec((NHEADS, 2 * NHID), lambda c, i: (0, 0)),
        ],
        out_specs=[
            pl.BlockSpec((BN, H1W), lambda c, i: (c * (N // BN) + i, 0)),
            pl.BlockSpec((BN, 8), lambda c, i: (i, 0)),
        ],
        out_shape=[
            jax.ShapeDtypeStruct((2 * N, H1W), jnp.float32),
            jax.ShapeDtypeStruct((N, 8), jnp.float32),
        ],
    )(x, W, a)


def _mm2_body(acc_ref, wout_ref, aout_ref, h2_ref, f2_ref):
    parts = []
    for i in range(NHEADS):
        c, hh = divmod(i, 2)
        v = acc_ref[c, :, hh * 64:(hh + 1) * 64]
        rs = acc_ref[c, :, 128 + hh][:, None]
        parts.append(_elu(v / (rs + 1e-16)))
    x2b = jnp.concatenate(parts, axis=1)
    h2_ref[:, :64] = jnp.dot(x2b, wout_ref[...], preferred_element_type=jnp.float32)
    h2_ref[:, 64:] = jnp.zeros((BN, H2W - 64), jnp.float32)
    wa2 = jnp.stack(
        [jnp.dot(wout_ref[...], aout_ref[:NCLASS],
                 preferred_element_type=jnp.float32),
         jnp.dot(wout_ref[...], aout_ref[NCLASS:],
                 preferred_element_type=jnp.float32)]
        + [jnp.zeros((NHEADS * NHID,), jnp.float32)] * 6, axis=1)
    f2_ref[...] = jnp.dot(x2b, wa2, preferred_element_type=jnp.float32)


def _mm2(acc1, w_out, a_out):
    return pl.pallas_call(
        _mm2_body,
        grid=(N // BN,),
        in_specs=[
            pl.BlockSpec((2, BN, H1W), lambda i: (0, i, 0)),
            pl.BlockSpec((NHEADS * NHID, NCLASS), lambda i: (0, 0)),
            pl.BlockSpec((2 * NCLASS,), lambda i: (0,)),
        ],
        out_specs=[
            pl.BlockSpec((BN, H2W), lambda i: (i, 0)),
            pl.BlockSpec((BN, 8), lambda i: (i, 0)),
        ],
        out_shape=[
            jax.ShapeDtypeStruct((N, H2W), jnp.float32),
            jax.ShapeDtypeStruct((N, 8), jnp.float32),
        ],
    )(acc1, w_out, a_out)


def _fin_body(acc_ref, out_ref):
    s = acc_ref[0, :, :64] + acc_ref[1, :, :64]
    rs = (acc_ref[0, :, 64] + acc_ref[1, :, 64])[:, None]
    out_ref[...] = _elu(s / (rs + 1e-16))


def _fin(acc2):
    return pl.pallas_call(
        _fin_body,
        grid=(N // BN,),
        in_specs=[pl.BlockSpec((2, BN, H2W), lambda i: (0, i, 0))],
        out_specs=pl.BlockSpec((BN, NCLASS), lambda i: (i, 0)),
        out_shape=jax.ShapeDtypeStruct((N, NCLASS), jnp.float32),
    )(acc2)


NEDGE = 320000
CHUNK = 80
NSUB = 16
ROWS_PER_SUB = N // NSUB
NBLK = CHUNK // 16
WSPLIT = [(k * 80, 80) for k in range(7)] + [(560, 65)]


def _splat_i32(x):
    return jnp.zeros((16,), jnp.int32) + x


def _zero_acc(zbuf, acc_sh, s, width):
    offs = [g * 16 for g in range(width // 16)]
    if width % 16:
        offs.append(width - 16)

    def zrow(r, _):
        for off in offs:
            zbuf[r, pl.ds(off, 16)] = jnp.zeros((16,), jnp.float32)
        return 0
    lax.fori_loop(0, CHUNK, zrow, 0)
    for off, nr in WSPLIT:
        pltpu.sync_copy(zbuf.at[pl.ds(0, nr)],
                        acc_sh.at[pl.ds(s * ROWS_PER_SUB + off, nr)])
    plsc.subcore_barrier()


def _writeout(acc_sh, out_hbm, c, s):
    plsc.subcore_barrier()
    for off, nr in WSPLIT:
        r0 = s * ROWS_PER_SUB + off
        pltpu.sync_copy(acc_sh.at[pl.ds(r0, nr)], out_hbm.at[c, pl.ds(r0, nr)])


def _edge_kernel1(adj_hbm, h_hbm, f_hbm, out_hbm,
                  acc_sh, sidx0, sidx1, sidx2, didx0, didx1, didx2,
                  gidx0, gidx1, gidx2, ssidx0, ssidx1, ssidx2,
                  fs0, fs1, fs2, fd0, fd1, fd2, w_v, rows0, rows1, rows2,
                  semg0, semg1, semg2, semi0, semi1, semi2,
                  sems0, sems1, sems2):
    c = lax.axis_index("c")
    s = lax.axis_index("s")
    sidx = (sidx0, sidx1, sidx2)
    didx = (didx0, didx1, didx2)
    gidx = (gidx0, gidx1, gidx2)
    ssidx = (ssidx0, ssidx1, ssidx2)
    fsb = (fs0, fs1, fs2)
    fdb = (fd0, fd1, fd2)
    rows = (rows0, rows1, rows2)
    semg = (semg0, semg1, semg2)
    semi = (semi0, semi1, semi2)
    sems = (sems0, sems1, sems2)
    lanes = lax.iota(jnp.int32, 16)

    _zero_acc(rows0, acc_sh, s, H1W)

    ep_tile = NEDGE // NSUB
    base_t = s * ep_tile
    nch = ep_tile // CHUNK

    def idx_load(b, g):
        base = base_t + g * CHUNK
        pltpu.async_copy(adj_hbm.at[0, pl.ds(base, CHUNK)], sidx[b], semi[b])
        pltpu.async_copy(adj_hbm.at[1, pl.ds(base, CHUNK)], didx[b], semi[b])

    def idx_drain(b):
        pltpu.make_async_copy(adj_hbm.at[0, pl.ds(0, CHUNK)], sidx[b], semi[b]).wait()
        pltpu.make_async_copy(adj_hbm.at[1, pl.ds(0, CHUNK)], didx[b], semi[b]).wait()

    def fire_gather(b):
        for blk in range(NBLK):
            gidx[b][pl.ds(blk * 16, 16)] = didx[b][pl.ds(blk * 16, 16)] + c * N
        pltpu.async_copy(h_hbm.at[gidx[b]], rows[b], semg[b])
        pltpu.async_copy(f_hbm.at[sidx[b]], fsb[b], semg[b])
        pltpu.async_copy(f_hbm.at[didx[b]], fdb[b], semg[b])

    def gather_drain(b):
        pltpu.make_async_copy(h_hbm.at[gidx[b]], rows[b], semg[b]).wait()
        pltpu.make_async_copy(f_hbm.at[sidx[b]], fsb[b], semg[b]).wait()
        pltpu.make_async_copy(f_hbm.at[didx[b]], fdb[b], semg[b]).wait()

    def scatter_drain(b):
        pltpu.make_async_copy(rows[b], acc_sh.at[ssidx[b]], sems[b]).wait()

    def process(b):
        for blk in range(NBLK):
            ssidx[b][pl.ds(blk * 16, 16)] = sidx[b][pl.ds(blk * 16, 16)]
            for hh in range(2):
                head = c * 2 + hh
                fs = plsc.load_gather(fsb[b], [lanes + blk * 16, _splat_i32(head)])
                fd = plsc.load_gather(fdb[b], [lanes + blk * 16, _splat_i32(head + 4)])
                z = fs + fd
                w = jnp.exp(-jnp.where(z >= 0, z, ALPHA * z))
                w_v[pl.ds(hh * CHUNK + blk * 16, 16)] = w

        @plsc.parallel_loop(0, CHUNK, unroll=8)
        def _scale(e):
            wv0 = plsc.load_gather(w_v, [_splat_i32(e)])
            wv1 = plsc.load_gather(w_v, [_splat_i32(e + CHUNK)])
            for gg in range(4):
                rows[b][e, pl.ds(gg * 16, 16)] = rows[b][e, pl.ds(gg * 16, 16)] * wv0
            for gg in range(4, 8):
                rows[b][e, pl.ds(gg * 16, 16)] = rows[b][e, pl.ds(gg * 16, 16)] * wv1

        for blk in range(NBLK):
            plsc.store_scatter(rows[b], [lanes + blk * 16, _splat_i32(128)],
                               w_v[pl.ds(blk * 16, 16)])
            plsc.store_scatter(rows[b], [lanes + blk * 16, _splat_i32(129)],
                               w_v[pl.ds(CHUNK + blk * 16, 16)])

    def chunk_step(g, cur, fire_next, load_next, static_g):
        nxt = (cur + 1) % 3
        if fire_next:
            idx_drain(nxt)
            if static_g:
                scatter_drain(nxt)
            else:
                @pl.when(g >= 2)
                def _():
                    scatter_drain(nxt)
            fire_gather(nxt)
        gather_drain(cur)
        process(cur)
        pltpu.async_copy(rows[cur], acc_sh.at[ssidx[cur]], sems[cur], add=True)
        if load_next:
            idx_load((cur + 2) % 3, g + 2)

    idx_load(0, 0)
    idx_drain(0)
    fire_gather(0)
    idx_load(1, 1)

    nfull = ((nch - 2) // 3) * 3

    def outer(go, _):
        for j in range(3):
            chunk_step(go * 3 + j, j, True, True, False)
        return 0
    lax.fori_loop(0, nfull // 3, outer, 0)
    for g in range(nfull, nch):
        chunk_step(g, g % 3, g + 1 < nch, g + 2 < nch, True)
    scatter_drain((nch - 3) % 3)
    scatter_drain((nch - 2) % 3)
    scatter_drain((nch - 1) % 3)

    _writeout(acc_sh, out_hbm, c, s)


def _edge1_sc(adj2, haug, f):
    mesh = plsc.VectorSubcoreMesh(core_axis_name="c", subcore_axis_name="s",
                                  num_cores=2, num_subcores=NSUB)
    idx_t = pltpu.VMEM((CHUNK,), jnp.int32)
    f_t = pltpu.VMEM((CHUNK, 8), jnp.float32)
    rows_t = pltpu.VMEM((CHUNK, H1W), jnp.float32)
    run = pl.kernel(
        _edge_kernel1,
        mesh=mesh,
        compiler_params=pltpu.CompilerParams(needs_layout_passes=False,
                                             use_tc_tiling_on_sc=False),
        out_type=jax.ShapeDtypeStruct((2, N, H1W), jnp.float32),
        scratch_types=(
            [pltpu.VMEM_SHARED((N, H1W), jnp.float32)]
            + [idx_t] * 12 + [f_t] * 6
            + [pltpu.VMEM((2 * CHUNK,), jnp.float32)]
            + [rows_t] * 3
            + [pltpu.SemaphoreType.DMA] * 9
        ),
    )
    return run(adj2, haug, f)


def _edge_kernel2(adj_hbm, h_hbm, f_hbm, out_hbm,
                  acc_sh, f_v, sidx0, sidx1, sidx2, didx0, didx1, didx2,
                  ssidx0, ssidx1, ssidx2, w_v, rows0, rows1, rows2,
                  semg0, semg1, semg2, semi0, semi1, semi2,
                  sems0, sems1, sems2):
    c = lax.axis_index("c")
    s = lax.axis_index("s")
    sidx = (sidx0, sidx1, sidx2)
    didx = (didx0, didx1, didx2)
    ssidx = (ssidx0, ssidx1, ssidx2)
    rows = (rows0, rows1, rows2)
    semg = (semg0, semg1, semg2)
    semi = (semi0, semi1, semi2)
    sems = (sems0, sems1, sems2)
    lanes = lax.iota(jnp.int32, 16)

    _zero_acc(rows0, acc_sh, s, H2W)

    pltpu.sync_copy(f_hbm, f_v)

    ep_w = NEDGE // (2 * NSUB)
    base_t = c * (NEDGE // 2) + s * ep_w
    nch = ep_w // CHUNK

    def idx_load(b, g):
        base = base_t + g * CHUNK
        pltpu.async_copy(adj_hbm.at[0, pl.ds(base, CHUNK)], sidx[b], semi[b])
        pltpu.async_copy(adj_hbm.at[1, pl.ds(base, CHUNK)], didx[b], semi[b])

    def idx_drain(b):
        pltpu.make_async_copy(adj_hbm.at[0, pl.ds(0, CHUNK)], sidx[b], semi[b]).wait()
        pltpu.make_async_copy(adj_hbm.at[1, pl.ds(0, CHUNK)], didx[b], semi[b]).wait()

    def fire_gather(b):
        pltpu.async_copy(h_hbm.at[didx[b]], rows[b], semg[b])

    def gather_drain(b):
        pltpu.make_async_copy(h_hbm.at[didx[b]], rows[b], semg[b]).wait()

    def scatter_drain(b):
        pltpu.make_async_copy(rows[b], acc_sh.at[ssidx[b]], sems[b]).wait()

    def process(b):
        for blk in range(NBLK):
            ssidx[b][pl.ds(blk * 16, 16)] = sidx[b][pl.ds(blk * 16, 16)]
            sv = sidx[b][pl.ds(blk * 16, 16)]
            dv = didx[b][pl.ds(blk * 16, 16)]
            fs = plsc.load_gather(f_v, [sv * 2])
            fd = plsc.load_gather(f_v, [dv * 2 + 1])
            z = fs + fd
            w_v[pl.ds(blk * 16, 16)] = jnp.exp(-jnp.where(z >= 0, z, ALPHA * z))

        @plsc.parallel_loop(0, CHUNK, unroll=8)
        def _scale(e):
            wv = plsc.load_gather(w_v, [_splat_i32(e)])
            for gg in range(4):
                rows[b][e, pl.ds(gg * 16, 16)] = rows[b][e, pl.ds(gg * 16, 16)] * wv

        for blk in range(NBLK):
            plsc.store_scatter(rows[b], [lanes + blk * 16, _splat_i32(64)],
                               w_v[pl.ds(blk * 16, 16)])

    def chunk_step(g, cur, fire_next, load_next, static_g):
        nxt = (cur + 1) % 3
        if fire_next:
            idx_drain(nxt)
            if static_g:
                scatter_drain(nxt)
            else:
                @pl.when(g >= 2)
                def _():
                    scatter_drain(nxt)
            fire_gather(nxt)
        gather_drain(cur)
        process(cur)
        pltpu.async_copy(rows[cur], acc_sh.at[ssidx[cur]], sems[cur], add=True)
        if load_next:
            idx_load((cur + 2) % 3, g + 2)

    idx_load(0, 0)
    idx_drain(0)
    fire_gather(0)
    idx_load(1, 1)

    nfull = ((nch - 2) // 3) * 3

    def outer(go, _):
        for j in range(3):
            chunk_step(go * 3 + j, j, True, True, False)
        return 0
    lax.fori_loop(0, nfull // 3, outer, 0)
    for g in range(nfull, nch):
        chunk_step(g, g % 3, g + 1 < nch, g + 2 < nch, True)
    scatter_drain((nch - 3) % 3)
    scatter_drain((nch - 2) % 3)
    scatter_drain((nch - 1) % 3)

    _writeout(acc_sh, out_hbm, c, s)


def _edge2_sc(adj2, h2aug, f2_flat):
    mesh = plsc.VectorSubcoreMesh(core_axis_name="c", subcore_axis_name="s",
                                  num_cores=2, num_subcores=NSUB)
    idx_t = pltpu.VMEM((CHUNK,), jnp.int32)
    rows_t = pltpu.VMEM((CHUNK, H2W), jnp.float32)
    run = pl.kernel(
        _edge_kernel2,
        mesh=mesh,
        compiler_params=pltpu.CompilerParams(needs_layout_passes=False,
                                             use_tc_tiling_on_sc=False),
        out_type=jax.ShapeDtypeStruct((2, N, H2W), jnp.float32),
        scratch_types=(
            [pltpu.VMEM_SHARED((N, H2W), jnp.float32),
             pltpu.VMEM((2 * N,), jnp.float32)]
            + [idx_t] * 9
            + [pltpu.VMEM((CHUNK,), jnp.float32)]
            + [rows_t] * 3
            + [pltpu.SemaphoreType.DMA] * 9
        ),
    )
    return run(adj2, h2aug, f2_flat)


def kernel(adj, x, W, a, W_out, a_out):
    adj2 = adj.astype(jnp.int32)

    h, f = _mm1(x, W, a)
    acc1 = _edge1_sc(adj2, h, f)
    h2, f2 = _mm2(acc1, W_out, a_out)
    acc2 = _edge2_sc(adj2, h2, f2[:, :2].reshape(-1))
    return _fin(acc2)


# speedup vs baseline: 1.0045x; 1.0045x over previous
import jax
import jax.numpy as jnp
from jax import lax
from jax.experimental import pallas as pl
from jax.experimental.pallas import tpu as pltpu
from jax.experimental.pallas import tpu_sc as plsc

N = 10000
NFEAT = 128
NHID = 64
NCLASS = 64
NHEADS = 4
ALPHA = 0.2

BN = 2000
H1W = 136
H2W = 72


def _elu(z):
    return jnp.where(z > 0, z, jnp.exp(z) - 1.0)


def _mm1_body(x_ref, w_ref, a_ref, h_ref, f_ref):
    c = pl.program_id(0)
    xb = x_ref[...]
    cols = [jnp.dot(w_ref[j], a_ref[j, :NHID],
                    preferred_element_type=jnp.float32) for j in range(NHEADS)]
    cols += [jnp.dot(w_ref[j], a_ref[j, NHID:],
                     preferred_element_type=jnp.float32) for j in range(NHEADS)]
    wfull = jnp.concatenate(
        [w_ref[2 * c], w_ref[2 * c + 1], jnp.stack(cols, axis=1)], axis=1)
    hb = jnp.dot(xb, wfull, preferred_element_type=jnp.float32)
    h_ref[...] = hb
    f_ref[...] = hb[:, 128:]


def _mm1(x, W, a):
    return pl.pallas_call(
        _mm1_body,
        grid=(2, N // BN),
        in_specs=[
            pl.BlockSpec((BN, NFEAT), lambda c, i: (i, 0)),
            pl.BlockSpec((NHEADS, NFEAT, NHID), lambda c, i: (0, 0, 0)),
            pl.BlockSpec((NHEADS, 2 * NHID), lambda c, i: (0, 0)),
        ],
        out_specs=[
            pl.BlockSpec((BN, H1W), lambda c, i: (c * (N // BN) + i, 0)),
            pl.BlockSpec((BN, 8), lambda c, i: (i, 0)),
        ],
        out_shape=[
            jax.ShapeDtypeStruct((2 * N, H1W), jnp.float32),
            jax.ShapeDtypeStruct((N, 8), jnp.float32),
        ],
    )(x, W, a)


def _mm2_body(acc_ref, wout_ref, aout_ref, h2_ref, f2_ref):
    parts = []
    for i in range(NHEADS):
        c, hh = divmod(i, 2)
        v = acc_ref[c, :, hh * 64:(hh + 1) * 64]
        rs = acc_ref[c, :, 128 + hh][:, None]
        parts.append(_elu(v / (rs + 1e-16)))
    x2b = jnp.concatenate(parts, axis=1)
    h2_ref[:, :64] = jnp.dot(x2b, wout_ref[...], preferred_element_type=jnp.float32)
    h2_ref[:, 64:] = jnp.zeros((BN, H2W - 64), jnp.float32)
    wa2 = jnp.stack(
        [jnp.dot(wout_ref[...], aout_ref[:NCLASS],
                 preferred_element_type=jnp.float32),
         jnp.dot(wout_ref[...], aout_ref[NCLASS:],
                 preferred_element_type=jnp.float32)]
        + [jnp.zeros((NHEADS * NHID,), jnp.float32)] * 6, axis=1)
    f2_ref[...] = jnp.dot(x2b, wa2, preferred_element_type=jnp.float32)


def _mm2(acc1, w_out, a_out):
    return pl.pallas_call(
        _mm2_body,
        grid=(N // BN,),
        in_specs=[
            pl.BlockSpec((2, BN, H1W), lambda i: (0, i, 0)),
            pl.BlockSpec((NHEADS * NHID, NCLASS), lambda i: (0, 0)),
            pl.BlockSpec((2 * NCLASS,), lambda i: (0,)),
        ],
        out_specs=[
            pl.BlockSpec((BN, H2W), lambda i: (i, 0)),
            pl.BlockSpec((BN, 8), lambda i: (i, 0)),
        ],
        out_shape=[
            jax.ShapeDtypeStruct((N, H2W), jnp.float32),
            jax.ShapeDtypeStruct((N, 8), jnp.float32),
        ],
    )(acc1, w_out, a_out)


def _fin_body(acc_ref, out_ref):
    s = acc_ref[0, :, :64] + acc_ref[1, :, :64]
    rs = (acc_ref[0, :, 64] + acc_ref[1, :, 64])[:, None]
    out_ref[...] = _elu(s / (rs + 1e-16))


def _fin(acc2):
    return pl.pallas_call(
        _fin_body,
        grid=(N // BN,),
        in_specs=[pl.BlockSpec((2, BN, H2W), lambda i: (0, i, 0))],
        out_specs=pl.BlockSpec((BN, NCLASS), lambda i: (i, 0)),
        out_shape=jax.ShapeDtypeStruct((N, NCLASS), jnp.float32),
    )(acc2)


NEDGE = 320000
CHUNK = 80
NSUB = 16
ROWS_PER_SUB = N // NSUB
NBLK = CHUNK // 16
WSPLIT = [(k * 80, 80) for k in range(7)] + [(560, 65)]


def _splat_i32(x):
    return jnp.zeros((16,), jnp.int32) + x


def _zero_acc(zbuf, acc_sh, s, width):
    offs = [g * 16 for g in range(width // 16)]
    if width % 16:
        offs.append(width - 16)

    def zrow(r, _):
        for off in offs:
            zbuf[r, pl.ds(off, 16)] = jnp.zeros((16,), jnp.float32)
        return 0
    lax.fori_loop(0, CHUNK, zrow, 0)
    for off, nr in WSPLIT:
        pltpu.sync_copy(zbuf.at[pl.ds(0, nr)],
                        acc_sh.at[pl.ds(s * ROWS_PER_SUB + off, nr)])
    plsc.subcore_barrier()


def _writeout(acc_sh, out_hbm, c, s):
    plsc.subcore_barrier()
    for off, nr in WSPLIT:
        r0 = s * ROWS_PER_SUB + off
        pltpu.sync_copy(acc_sh.at[pl.ds(r0, nr)], out_hbm.at[c, pl.ds(r0, nr)])


def _edge_kernel1(adj_hbm, h_hbm, f_hbm, out_hbm,
                  acc_sh, sidx0, sidx1, sidx2, didx0, didx1, didx2,
                  gidx0, gidx1, gidx2, ssidx0, ssidx1, ssidx2,
                  fs0, fs1, fs2, fd0, fd1, fd2, w_v, rows0, rows1, rows2,
                  semg0, semg1, semg2, semi0, semi1, semi2,
                  sems0, sems1, sems2):
    c = lax.axis_index("c")
    s = lax.axis_index("s")
    sidx = (sidx0, sidx1, sidx2)
    didx = (didx0, didx1, didx2)
    gidx = (gidx0, gidx1, gidx2)
    ssidx = (ssidx0, ssidx1, ssidx2)
    fsb = (fs0, fs1, fs2)
    fdb = (fd0, fd1, fd2)
    rows = (rows0, rows1, rows2)
    semg = (semg0, semg1, semg2)
    semi = (semi0, semi1, semi2)
    sems = (sems0, sems1, sems2)
    lanes = lax.iota(jnp.int32, 16)

    _zero_acc(rows0, acc_sh, s, H1W)

    ep_tile = NEDGE // NSUB
    base_t = s * ep_tile
    nch = ep_tile // CHUNK

    def idx_load(b, g):
        base = base_t + g * CHUNK
        pltpu.async_copy(adj_hbm.at[0, pl.ds(base, CHUNK)], sidx[b], semi[b])
        pltpu.async_copy(adj_hbm.at[1, pl.ds(base, CHUNK)], didx[b], semi[b])

    def idx_drain(b):
        pltpu.make_async_copy(adj_hbm.at[0, pl.ds(0, CHUNK)], sidx[b], semi[b]).wait()
        pltpu.make_async_copy(adj_hbm.at[1, pl.ds(0, CHUNK)], didx[b], semi[b]).wait()

    def fire_gather(b):
        for blk in range(NBLK):
            gidx[b][pl.ds(blk * 16, 16)] = didx[b][pl.ds(blk * 16, 16)] + c * N
        pltpu.async_copy(h_hbm.at[gidx[b]], rows[b], semg[b])
        pltpu.async_copy(f_hbm.at[sidx[b]], fsb[b], semg[b])
        pltpu.async_copy(f_hbm.at[didx[b]], fdb[b], semg[b])

    def gather_drain(b):
        pltpu.make_async_copy(h_hbm.at[gidx[b]], rows[b], semg[b]).wait()
        pltpu.make_async_copy(f_hbm.at[sidx[b]], fsb[b], semg[b]).wait()
        pltpu.make_async_copy(f_hbm.at[didx[b]], fdb[b], semg[b]).wait()

    def scatter_drain(b):
        pltpu.make_async_copy(rows[b], acc_sh.at[ssidx[b]], sems[b]).wait()

    def process(b):
        for blk in range(NBLK):
            ssidx[b][pl.ds(blk * 16, 16)] = sidx[b][pl.ds(blk * 16, 16)]
            for hh in range(2):
                head = c * 2 + hh
                fs = plsc.load_gather(fsb[b], [lanes + blk * 16, _splat_i32(head)])
                fd = plsc.load_gather(fdb[b], [lanes + blk * 16, _splat_i32(head + 4)])
                z = fs + fd
                w = jnp.exp(-jnp.where(z >= 0, z, ALPHA * z))
                w_v[pl.ds(hh * CHUNK + blk * 16, 16)] = w

        @plsc.parallel_loop(0, CHUNK, unroll=4)
        def _scale(e):
            wv0 = plsc.load_gather(w_v, [_splat_i32(e)])
            wv1 = plsc.load_gather(w_v, [_splat_i32(e + CHUNK)])
            for gg in range(4):
                rows[b][e, pl.ds(gg * 16, 16)] = rows[b][e, pl.ds(gg * 16, 16)] * wv0
            for gg in range(4, 8):
                rows[b][e, pl.ds(gg * 16, 16)] = rows[b][e, pl.ds(gg * 16, 16)] * wv1

        for blk in range(NBLK):
            plsc.store_scatter(rows[b], [lanes + blk * 16, _splat_i32(128)],
                               w_v[pl.ds(blk * 16, 16)])
            plsc.store_scatter(rows[b], [lanes + blk * 16, _splat_i32(129)],
                               w_v[pl.ds(CHUNK + blk * 16, 16)])

    def chunk_step(g, cur, fire_next, load_next, static_g):
        nxt = (cur + 1) % 3
        if fire_next:
            idx_drain(nxt)
            if static_g:
                scatter_drain(nxt)
            else:
                @pl.when(g >= 2)
                def _():
                    scatter_drain(nxt)
            fire_gather(nxt)
        gather_drain(cur)
        process(cur)
        pltpu.async_copy(rows[cur], acc_sh.at[ssidx[cur]], sems[cur], add=True)
        if load_next:
            idx_load((cur + 2) % 3, g + 2)

    idx_load(0, 0)
    idx_drain(0)
    fire_gather(0)
    idx_load(1, 1)

    nfull = ((nch - 2) // 3) * 3

    def outer(go, _):
        for j in range(3):
            chunk_step(go * 3 + j, j, True, True, False)
        return 0
    lax.fori_loop(0, nfull // 3, outer, 0)
    for g in range(nfull, nch):
        chunk_step(g, g % 3, g + 1 < nch, g + 2 < nch, True)
    scatter_drain((nch - 3) % 3)
    scatter_drain((nch - 2) % 3)
    scatter_drain((nch - 1) % 3)

    _writeout(acc_sh, out_hbm, c, s)


def _edge1_sc(adj2, haug, f):
    mesh = plsc.VectorSubcoreMesh(core_axis_name="c", subcore_axis_name="s",
                                  num_cores=2, num_subcores=NSUB)
    idx_t = pltpu.VMEM((CHUNK,), jnp.int32)
    f_t = pltpu.VMEM((CHUNK, 8), jnp.float32)
    rows_t = pltpu.VMEM((CHUNK, H1W), jnp.float32)
    run = pl.kernel(
        _edge_kernel1,
        mesh=mesh,
        compiler_params=pltpu.CompilerParams(needs_layout_passes=False,
                                             use_tc_tiling_on_sc=False),
        out_type=jax.ShapeDtypeStruct((2, N, H1W), jnp.float32),
        scratch_types=(
            [pltpu.VMEM_SHARED((N, H1W), jnp.float32)]
            + [idx_t] * 12 + [f_t] * 6
            + [pltpu.VMEM((2 * CHUNK,), jnp.float32)]
            + [rows_t] * 3
            + [pltpu.SemaphoreType.DMA] * 9
        ),
    )
    return run(adj2, haug, f)


def _edge_kernel2(adj_hbm, h_hbm, f_hbm, out_hbm,
                  acc_sh, f_v, sidx0, sidx1, sidx2, didx0, didx1, didx2,
                  ssidx0, ssidx1, ssidx2, w_v, rows0, rows1, rows2,
                  semg0, semg1, semg2, semi0, semi1, semi2,
                  sems0, sems1, sems2):
    c = lax.axis_index("c")
    s = lax.axis_index("s")
    sidx = (sidx0, sidx1, sidx2)
    didx = (didx0, didx1, didx2)
    ssidx = (ssidx0, ssidx1, ssidx2)
    rows = (rows0, rows1, rows2)
    semg = (semg0, semg1, semg2)
    semi = (semi0, semi1, semi2)
    sems = (sems0, sems1, sems2)
    lanes = lax.iota(jnp.int32, 16)

    _zero_acc(rows0, acc_sh, s, H2W)

    pltpu.sync_copy(f_hbm, f_v)

    ep_w = NEDGE // (2 * NSUB)
    base_t = c * (NEDGE // 2) + s * ep_w
    nch = ep_w // CHUNK

    def idx_load(b, g):
        base = base_t + g * CHUNK
        pltpu.async_copy(adj_hbm.at[0, pl.ds(base, CHUNK)], sidx[b], semi[b])
        pltpu.async_copy(adj_hbm.at[1, pl.ds(base, CHUNK)], didx[b], semi[b])

    def idx_drain(b):
        pltpu.make_async_copy(adj_hbm.at[0, pl.ds(0, CHUNK)], sidx[b], semi[b]).wait()
        pltpu.make_async_copy(adj_hbm.at[1, pl.ds(0, CHUNK)], didx[b], semi[b]).wait()

    def fire_gather(b):
        pltpu.async_copy(h_hbm.at[didx[b]], rows[b], semg[b])

    def gather_drain(b):
        pltpu.make_async_copy(h_hbm.at[didx[b]], rows[b], semg[b]).wait()

    def scatter_drain(b):
        pltpu.make_async_copy(rows[b], acc_sh.at[ssidx[b]], sems[b]).wait()

    def process(b):
        for blk in range(NBLK):
            ssidx[b][pl.ds(blk * 16, 16)] = sidx[b][pl.ds(blk * 16, 16)]
            sv = sidx[b][pl.ds(blk * 16, 16)]
            dv = didx[b][pl.ds(blk * 16, 16)]
            fs = plsc.load_gather(f_v, [sv * 2])
            fd = plsc.load_gather(f_v, [dv * 2 + 1])
            z = fs + fd
            w_v[pl.ds(blk * 16, 16)] = jnp.exp(-jnp.where(z >= 0, z, ALPHA * z))

        @plsc.parallel_loop(0, CHUNK, unroll=4)
        def _scale(e):
            wv = plsc.load_gather(w_v, [_splat_i32(e)])
            for gg in range(4):
                rows[b][e, pl.ds(gg * 16, 16)] = rows[b][e, pl.ds(gg * 16, 16)] * wv

        for blk in range(NBLK):
            plsc.store_scatter(rows[b], [lanes + blk * 16, _splat_i32(64)],
                               w_v[pl.ds(blk * 16, 16)])

    def chunk_step(g, cur, fire_next, load_next, static_g):
        nxt = (cur + 1) % 3
        if fire_next:
            idx_drain(nxt)
            if static_g:
                scatter_drain(nxt)
            else:
                @pl.when(g >= 2)
                def _():
                    scatter_drain(nxt)
            fire_gather(nxt)
        gather_drain(cur)
        process(cur)
        pltpu.async_copy(rows[cur], acc_sh.at[ssidx[cur]], sems[cur], add=True)
        if load_next:
            idx_load((cur + 2) % 3, g + 2)

    idx_load(0, 0)
    idx_drain(0)
    fire_gather(0)
    idx_load(1, 1)

    nfull = ((nch - 2) // 3) * 3

    def outer(go, _):
        for j in range(3):
            chunk_step(go * 3 + j, j, True, True, False)
        return 0
    lax.fori_loop(0, nfull // 3, outer, 0)
    for g in range(nfull, nch):
        chunk_step(g, g % 3, g + 1 < nch, g + 2 < nch, True)
    scatter_drain((nch - 3) % 3)
    scatter_drain((nch - 2) % 3)
    scatter_drain((nch - 1) % 3)

    _writeout(acc_sh, out_hbm, c, s)


def _edge2_sc(adj2, h2aug, f2_flat):
    mesh = plsc.VectorSubcoreMesh(core_axis_name="c", subcore_axis_name="s",
                                  num_cores=2, num_subcores=NSUB)
    idx_t = pltpu.VMEM((CHUNK,), jnp.int32)
    rows_t = pltpu.VMEM((CHUNK, H2W), jnp.float32)
    run = pl.kernel(
        _edge_kernel2,
        mesh=mesh,
        compiler_params=pltpu.CompilerParams(needs_layout_passes=False,
                                             use_tc_tiling_on_sc=False),
        out_type=jax.ShapeDtypeStruct((2, N, H2W), jnp.float32),
        scratch_types=(
            [pltpu.VMEM_SHARED((N, H2W), jnp.float32),
             pltpu.VMEM((2 * N,), jnp.float32)]
            + [idx_t] * 9
            + [pltpu.VMEM((CHUNK,), jnp.float32)]
            + [rows_t] * 3
            + [pltpu.SemaphoreType.DMA] * 9
        ),
    )
    return run(adj2, h2aug, f2_flat)


def kernel(adj, x, W, a, W_out, a_out):
    adj2 = adj.astype(jnp.int32)

    h, f = _mm1(x, W, a)
    acc1 = _edge1_sc(adj2, h, f)
    h2, f2 = _mm2(acc1, W_out, a_out)
    acc2 = _edge2_sc(adj2, h2, f2[:, :2].reshape(-1))
    return _fin(acc2)
